# Initial kernel scaffold; baseline (speedup 1.0000x reference)
import jax
import jax.numpy as jnp
from jax.experimental import pallas as pl


def kernel(latent_space, edge_index, W1, b1, W2, b2):
    raise NotImplementedError("write your pallas kernel here")


# speedup vs baseline: 2.7120x; 2.7120x over previous
import functools

import jax
import jax.numpy as jnp
from jax import lax
from jax.experimental import pallas as pl
from jax.experimental.pallas import tpu as pltpu
from jax.experimental.pallas import tpu_sc as plsc

_SC_CORES = 2
_SC_SUBCORES = 16
_LANES = 16
_CHUNK = 128


def _precompute_tables(latent, w1a, w1b, b1):
    n, d = latent.shape
    h = w1a.shape[1]
    blk = 2000
    dn = (((1,), (0,)), ((), ()))

    def body(lat_ref, w1a_ref, w1b_ref, b1_ref, a_ref, b_ref):
        x = lat_ref[...]
        a_ref[...] = (
            lax.dot_general(x, w1a_ref[...], dn, precision=lax.Precision.HIGHEST)
            + b1_ref[...]
        )
        b_ref[...] = lax.dot_general(
            x, w1b_ref[...], dn, precision=lax.Precision.HIGHEST
        )

    return pl.pallas_call(
        body,
        grid=(n // blk,),
        in_specs=[
            pl.BlockSpec((blk, d), lambda i: (i, 0)),
            pl.BlockSpec((d, h), lambda i: (0, 0)),
            pl.BlockSpec((d, h), lambda i: (0, 0)),
            pl.BlockSpec((1, h), lambda i: (0, 0)),
        ],
        out_specs=[
            pl.BlockSpec((blk, h), lambda i: (i, 0)),
            pl.BlockSpec((blk, h), lambda i: (i, 0)),
        ],
        out_shape=[jax.ShapeDtypeStruct((n, h), jnp.float32)] * 2,
    )(latent, w1a, w1b, b1.reshape(1, h))


def _sc_gather_add(table_a, table_b, src, dst):
    e = src.shape[0]
    h = table_a.shape[1]
    nw = _SC_CORES * _SC_SUBCORES
    n_chunks = e // _CHUNK
    per_worker = -(-n_chunks // nw)

    mesh = plsc.VectorSubcoreMesh(core_axis_name="c", subcore_axis_name="s")

    @functools.partial(
        pl.kernel,
        mesh=mesh,
        out_type=jax.ShapeDtypeStruct((e, h), jnp.float32),
        scratch_types=[
            pltpu.VMEM((_CHUNK,), jnp.int32),
            pltpu.VMEM((_CHUNK,), jnp.int32),
            pltpu.VMEM((_CHUNK, h), jnp.float32),
            pltpu.VMEM((_CHUNK, h), jnp.float32),
            pltpu.SemaphoreType.DMA,
            pltpu.SemaphoreType.DMA,
        ],
    )
    def k(a_hbm, b_hbm, src_hbm, dst_hbm, s_hbm, idx_s, idx_d, buf_a, buf_b,
          sem_a, sem_b):
        wid = lax.axis_index("s") * _SC_CORES + lax.axis_index("c")

        @pl.loop(0, per_worker)
        def _(i):
            cid = i * nw + wid

            @pl.when(cid < n_chunks)
            def _():
                base = cid * _CHUNK
                pltpu.sync_copy(src_hbm.at[pl.ds(base, _CHUNK)], idx_s)
                pltpu.sync_copy(dst_hbm.at[pl.ds(base, _CHUNK)], idx_d)
                cp_a = pltpu.async_copy(a_hbm.at[idx_s], buf_a, sem_a)
                cp_b = pltpu.async_copy(b_hbm.at[idx_d], buf_b, sem_b)
                cp_a.wait()
                cp_b.wait()

                @pl.loop(0, _CHUNK)
                def _(r):
                    for c in range(h // _LANES):
                        sl = pl.ds(c * _LANES, _LANES)
                        plsc.addupdate(buf_a.at[r, sl], buf_b[r, sl])

                pltpu.sync_copy(buf_a, s_hbm.at[pl.ds(base, _CHUNK)])

    return k(table_a, table_b, src, dst)


def _tc_tail(s, w2, b2):
    e, h = s.shape
    blk = 4000
    dn = (((1,), (0,)), ((), ()))

    def body(s_ref, w2_ref, b2_ref, o_ref):
        x = jnp.maximum(s_ref[...], 0.0)
        logit = lax.dot_general(
            x, w2_ref[...], dn, precision=lax.Precision.HIGHEST
        ) + b2_ref[0]
        o_ref[...] = jax.nn.sigmoid(logit)

    return pl.pallas_call(
        body,
        grid=(e // blk,),
        in_specs=[
            pl.BlockSpec((blk, h), lambda i: (i, 0)),
            pl.BlockSpec((h, 1), lambda i: (0, 0)),
            pl.BlockSpec(memory_space=pltpu.SMEM),
        ],
        out_specs=pl.BlockSpec((blk, 1), lambda i: (i, 0)),
        out_shape=jax.ShapeDtypeStruct((e, 1), jnp.float32),
    )(s, w2, b2)


def kernel(latent_space, edge_index, W1, b1, W2, b2):
    d = latent_space.shape[1]
    src = edge_index[0].astype(jnp.int32)
    dst = edge_index[1].astype(jnp.int32)
    table_a, table_b = _precompute_tables(latent_space, W1[:d], W1[d:], b1)
    s = _sc_gather_add(table_a, table_b, src, dst)
    out = _tc_tail(s, W2, b2)
    return out[:, 0]


# speedup vs baseline: 3.8920x; 1.4351x over previous
import functools

import jax
import jax.numpy as jnp
from jax import lax
from jax.experimental import pallas as pl
from jax.experimental.pallas import tpu as pltpu
from jax.experimental.pallas import tpu_sc as plsc

_SC_CORES = 2
_SC_SUBCORES = 16
_LANES = 16
_CHUNK = 128


def _precompute_tables(latent, w1a, w1b, b1):
    n, d = latent.shape
    h = w1a.shape[1]
    blk = 2000
    dn = (((1,), (0,)), ((), ()))

    def body(lat_ref, w1a_ref, w1b_ref, b1_ref, a_ref, b_ref):
        x = lat_ref[...]
        a_ref[...] = (
            lax.dot_general(x, w1a_ref[...], dn, precision=lax.Precision.HIGHEST)
            + b1_ref[...]
        )
        b_ref[...] = lax.dot_general(
            x, w1b_ref[...], dn, precision=lax.Precision.HIGHEST
        )

    return pl.pallas_call(
        body,
        grid=(n // blk,),
        in_specs=[
            pl.BlockSpec((blk, d), lambda i: (i, 0)),
            pl.BlockSpec((d, h), lambda i: (0, 0)),
            pl.BlockSpec((d, h), lambda i: (0, 0)),
            pl.BlockSpec((1, h), lambda i: (0, 0)),
        ],
        out_specs=[
            pl.BlockSpec((blk, h), lambda i: (i, 0)),
            pl.BlockSpec((blk, h), lambda i: (i, 0)),
        ],
        out_shape=[jax.ShapeDtypeStruct((n, h), jnp.float32)] * 2,
    )(latent, w1a, w1b, b1.reshape(1, h))


_NBUF = 3


def _sc_gather_add(table_a, table_b, src, dst):
    e = src.shape[0]
    h = table_a.shape[1]
    nw = _SC_CORES * _SC_SUBCORES
    n_chunks = e // _CHUNK
    per_worker = -(-n_chunks // nw)

    mesh = plsc.VectorSubcoreMesh(core_axis_name="c", subcore_axis_name="s")

    scratch = (
        [pltpu.VMEM((_CHUNK,), jnp.int32) for _ in range(2 * _NBUF)]
        + [pltpu.VMEM((_CHUNK, h), jnp.float32) for _ in range(2 * _NBUF)]
        + [pltpu.SemaphoreType.DMA for _ in range(2 * _NBUF)]
    )

    @functools.partial(
        pl.kernel,
        mesh=mesh,
        out_type=jax.ShapeDtypeStruct((e, h), jnp.float32),
        scratch_types=scratch,
    )
    def k(a_hbm, b_hbm, src_hbm, dst_hbm, s_hbm, *bufs):
        idx_s = bufs[0:_NBUF]
        idx_d = bufs[_NBUF:2 * _NBUF]
        buf_a = bufs[2 * _NBUF:3 * _NBUF]
        buf_b = bufs[3 * _NBUF:4 * _NBUF]
        sem_g = bufs[4 * _NBUF:5 * _NBUF]
        sem_o = bufs[5 * _NBUF:6 * _NBUF]

        wid = lax.axis_index("s") * _SC_CORES + lax.axis_index("c")
        nv = (n_chunks - 1 - wid) // nw + 1

        def prep(c, b):
            base = (c * nw + wid) * _CHUNK
            ci = pltpu.async_copy(src_hbm.at[pl.ds(base, _CHUNK)], idx_s[b],
                                  sem_g[b])
            cj = pltpu.async_copy(dst_hbm.at[pl.ds(base, _CHUNK)], idx_d[b],
                                  sem_g[b])
            ci.wait()
            cj.wait()
            pltpu.async_copy(a_hbm.at[idx_s[b]], buf_a[b], sem_g[b])
            pltpu.async_copy(b_hbm.at[idx_d[b]], buf_b[b], sem_g[b])

        def wait_gathers(b):
            pltpu.make_async_copy(a_hbm.at[idx_s[b]], buf_a[b], sem_g[b]).wait()
            pltpu.make_async_copy(b_hbm.at[idx_d[b]], buf_b[b], sem_g[b]).wait()

        def wait_out(b):
            pltpu.make_async_copy(buf_a[b], s_hbm.at[pl.ds(0, _CHUNK)],
                                  sem_o[b]).wait()

        prep(0, 0)

        @pl.loop(0, per_worker, step=_NBUF)
        def _(kk):
            for b in range(_NBUF):
                c = kk + b
                b1 = (b + 1) % _NBUF

                @pl.when(c + 1 < nv)
                def _():
                    if _NBUF > 1:
                        @pl.when(c + 1 >= _NBUF)
                        def _():
                            wait_out(b1)
                    prep(c + 1, b1)

                @pl.when(c < nv)
                def _():
                    wait_gathers(b)

                    @pl.loop(0, _CHUNK, step=2)
                    def _(r0):
                        for dr in range(2):
                            for cc in range(h // _LANES):
                                sl = pl.ds(cc * _LANES, _LANES)
                                plsc.addupdate(buf_a[b].at[r0 + dr, sl],
                                               buf_b[b][r0 + dr, sl])

                    base = (c * nw + wid) * _CHUNK
                    pltpu.async_copy(buf_a[b], s_hbm.at[pl.ds(base, _CHUNK)],
                                     sem_o[b])

        for b in range(_NBUF):
            @pl.when(nv > b)
            def _():
                wait_out(b)

    return k(table_a, table_b, src, dst)


def _tc_tail(s, w2, b2):
    e, h = s.shape
    blk = 4000
    dn = (((1,), (0,)), ((), ()))

    def body(s_ref, w2_ref, b2_ref, o_ref):
        x = jnp.maximum(s_ref[...], 0.0)
        logit = lax.dot_general(
            x, w2_ref[...], dn, precision=lax.Precision.HIGHEST
        ) + b2_ref[0]
        o_ref[...] = jax.nn.sigmoid(logit)

    return pl.pallas_call(
        body,
        grid=(e // blk,),
        in_specs=[
            pl.BlockSpec((blk, h), lambda i: (i, 0)),
            pl.BlockSpec((h, 1), lambda i: (0, 0)),
            pl.BlockSpec(memory_space=pltpu.SMEM),
        ],
        out_specs=pl.BlockSpec((blk, 1), lambda i: (i, 0)),
        out_shape=jax.ShapeDtypeStruct((e, 1), jnp.float32),
    )(s, w2, b2)


def kernel(latent_space, edge_index, W1, b1, W2, b2):
    d = latent_space.shape[1]
    src = edge_index[0].astype(jnp.int32)
    dst = edge_index[1].astype(jnp.int32)
    table_a, table_b = _precompute_tables(latent_space, W1[:d], W1[d:], b1)
    s = _sc_gather_add(table_a, table_b, src, dst)
    out = _tc_tail(s, W2, b2)
    return out[:, 0]
